# baseline (speedup 1.0000x reference)
import functools

import jax
import jax.numpy as jnp
from jax import lax
from jax.experimental import pallas as pl
from jax.experimental.pallas import tpu as pltpu
from jax.experimental.pallas import tpu_sc as plsc

NC, NS, L = 2, 16, 16
NW = NC * NS
E = 320000
D = 128
CHUNK = 128
CPW = 80
EPW = CPW * CHUNK
E_PAD = NW * EPW
SEGS = D // L
GROUPS = CHUNK // L

_mesh = plsc.VectorSubcoreMesh(core_axis_name="c", subcore_axis_name="s")


@functools.partial(
    pl.kernel,
    out_type=jax.ShapeDtypeStruct((NW, CPW, CHUNK), jnp.float32),
    mesh=_mesh,
    compiler_params=pltpu.CompilerParams(needs_layout_passes=False),
    scratch_types=[
        pltpu.VMEM((CPW, CHUNK), jnp.int32),
        pltpu.VMEM((CPW, CHUNK), jnp.int32),
        pltpu.VMEM((CHUNK, D), jnp.float32),
        pltpu.VMEM((CHUNK, D), jnp.float32),
        pltpu.VMEM((CHUNK, D), jnp.float32),
        pltpu.VMEM((CHUNK, D), jnp.float32),
        pltpu.VMEM((CPW, CHUNK), jnp.float32),
        pltpu.SemaphoreType.DMA,
        pltpu.SemaphoreType.DMA,
        pltpu.SemaphoreType.DMA,
        pltpu.SemaphoreType.DMA,
    ],
)
def _sc_dot(x_hbm, src_hbm, dst_hbm, out_hbm,
            sidx, didx, u_a, u_b, v_a, v_b, scores,
            sem_ua, sem_ub, sem_va, sem_vb):
    wid = lax.axis_index("s") * NC + lax.axis_index("c")
    lanes = lax.iota(jnp.int32, L)

    pltpu.sync_copy(src_hbm.at[wid], sidx)
    pltpu.sync_copy(dst_hbm.at[wid], didx)

    def issue(c, ubuf, vbuf, sem_u, sem_v):
        cu = pltpu.async_copy(x_hbm.at[sidx.at[c]], ubuf, sem_u)
        cv = pltpu.async_copy(x_hbm.at[didx.at[c]], vbuf, sem_v)
        return cu, cv

    def wait(ubuf, vbuf, sem_u, sem_v):
        pltpu.make_async_copy(x_hbm.at[sidx.at[0]], ubuf, sem_u).wait()
        pltpu.make_async_copy(x_hbm.at[didx.at[0]], vbuf, sem_v).wait()

    def compute(c, ubuf, vbuf):
        def group_body(g, gcarry):
            vec = jnp.zeros((L,), jnp.float32)
            for e in range(L):
                row = g * L + e
                acc = ubuf[row, pl.ds(0, L)] * vbuf[row, pl.ds(0, L)]
                for s in range(1, SEGS):
                    acc = acc + (ubuf[row, pl.ds(s * L, L)]
                                 * vbuf[row, pl.ds(s * L, L)])
                vec = jnp.where(lanes == e, jnp.sum(acc), vec)
            scores[c, pl.ds(g * L, L)] = vec
            return gcarry

        lax.fori_loop(0, GROUPS, group_body, 0)

    issue(0, u_a, v_a, sem_ua, sem_va)

    def pair_body(i, carry):
        c0 = 2 * i
        issue(c0 + 1, u_b, v_b, sem_ub, sem_vb)
        wait(u_a, v_a, sem_ua, sem_va)
        issue(c0 + 2, u_a, v_a, sem_ua, sem_va)
        wait(u_b, v_b, sem_ub, sem_vb)
        return carry

    lax.fori_loop(0, CPW // 2 - 1, pair_body, 0)
    issue(CPW - 1, u_b, v_b, sem_ub, sem_vb)
    wait(u_a, v_a, sem_ua, sem_va)
    wait(u_b, v_b, sem_ub, sem_vb)
    compute(0, u_a, v_a)
    compute(1, u_b, v_b)

    pltpu.sync_copy(scores, out_hbm.at[wid])


def kernel(x, edge_index):
    ei = edge_index.astype(jnp.int32)
    src = jnp.pad(ei[0], (0, E_PAD - E)).reshape(NW, CPW, CHUNK)
    dst = jnp.pad(ei[1], (0, E_PAD - E)).reshape(NW, CPW, CHUNK)
    return _sc_dot(x, src, dst).reshape(E_PAD)[:E]


# speedup vs baseline: 1.8127x; 1.8127x over previous
import functools

import jax
import jax.numpy as jnp
from jax import lax
from jax.experimental import pallas as pl
from jax.experimental.pallas import tpu as pltpu
from jax.experimental.pallas import tpu_sc as plsc

NC, NS, L = 2, 16, 16
E = 320000
N = 10000
N_PAD = 10240
D = 128
DW = D // 2
CHUNK = 128
CPT = 160
CPW0 = 40
CPW1 = CPT - CPW0
EPT = CPT * CHUNK
E_PAD = NS * EPT
SEGS = DW // L
GROUPS = CHUNK // L

_mesh = plsc.VectorSubcoreMesh(core_axis_name="c", subcore_axis_name="s")


@functools.partial(
    pl.kernel,
    out_type=jax.ShapeDtypeStruct((NS, EPT), jnp.float32),
    mesh=_mesh,
    compiler_params=pltpu.CompilerParams(needs_layout_passes=False,
                                         use_tc_tiling_on_sc=False),
    scratch_types=[
        pltpu.VMEM((CPT, CHUNK), jnp.int32),
        pltpu.VMEM((CHUNK,), jnp.int32),
        pltpu.VMEM((CHUNK,), jnp.int32),
        pltpu.VMEM((CHUNK,), jnp.int32),
        pltpu.VMEM((CHUNK,), jnp.int32),
        pltpu.VMEM((CHUNK, DW), jnp.int32),
        pltpu.VMEM((CHUNK, DW), jnp.int32),
        pltpu.VMEM((CHUNK, DW), jnp.int32),
        pltpu.VMEM((CHUNK, DW), jnp.int32),
        pltpu.VMEM((EPT,), jnp.float32),
        pltpu.SemaphoreType.DMA,
        pltpu.SemaphoreType.DMA,
        pltpu.SemaphoreType.DMA,
        pltpu.SemaphoreType.DMA,
    ],
)
def _sc_dot(x_hbm, pidx_hbm, out_hbm,
            pidx, su_a, su_b, sv_a, sv_b, u_a, u_b, v_a, v_b, scores,
            sem_ua, sem_ub, sem_va, sem_vb):
    cid = lax.axis_index("c")
    sid = lax.axis_index("s")
    lanes = lax.iota(jnp.int32, L)

    n = jnp.where(cid == 0, CPW0, CPW1)
    start = jnp.where(cid == 0, 0, CPW0)

    pltpu.sync_copy(pidx_hbm.at[sid], pidx)

    def unpack_idx(gc, subuf, svbuf):
        def body(g, carry):
            w = pidx[gc, pl.ds(g * L, L)]
            subuf[pl.ds(g * L, L)] = w & 0xFFFF
            svbuf[pl.ds(g * L, L)] = lax.shift_right_logical(w, 16)
            return carry
        lax.fori_loop(0, GROUPS, body, 0)

    def issue(subuf, svbuf, ubuf, vbuf, sem_u, sem_v):
        pltpu.async_copy(x_hbm.at[subuf], ubuf, sem_u)
        pltpu.async_copy(x_hbm.at[svbuf], vbuf, sem_v)

    def wait(subuf, svbuf, ubuf, vbuf, sem_u, sem_v):
        pltpu.make_async_copy(x_hbm.at[subuf], ubuf, sem_u).wait()
        pltpu.make_async_copy(x_hbm.at[svbuf], vbuf, sem_v).wait()

    def compute(lc, ubuf, vbuf):
        def group_body(g, gcarry):
            vec = jnp.zeros((L,), jnp.float32)
            for e in range(L):
                row = g * L + e
                acc = jnp.zeros((L,), jnp.float32)
                for s in range(SEGS):
                    useg = plsc.bitcast(ubuf[row, pl.ds(s * L, L)],
                                        jnp.bfloat16)
                    vseg = plsc.bitcast(vbuf[row, pl.ds(s * L, L)],
                                        jnp.bfloat16)
                    ua, ub = plsc.unpack(useg,
                                         format=plsc.PackFormat.INTERLEAVED)
                    va, vb = plsc.unpack(vseg,
                                         format=plsc.PackFormat.INTERLEAVED)
                    acc = acc + ua * va + ub * vb
                vec = jnp.where(lanes == e, jnp.sum(acc), vec)
            scores[pl.ds(lc * CHUNK + g * L, L)] = vec
            return gcarry

        lax.fori_loop(0, GROUPS, group_body, 0)

    unpack_idx(start, su_a, sv_a)
    issue(su_a, sv_a, u_a, v_a, sem_ua, sem_va)

    def pair_body(i, carry):
        lc0 = 2 * i
        unpack_idx(start + lc0 + 1, su_b, sv_b)
        issue(su_b, sv_b, u_b, v_b, sem_ub, sem_vb)
        wait(su_a, sv_a, u_a, v_a, sem_ua, sem_va)
        compute(lc0, u_a, v_a)
        unpack_idx(start + lc0 + 2, su_a, sv_a)
        issue(su_a, sv_a, u_a, v_a, sem_ua, sem_va)
        wait(su_b, sv_b, u_b, v_b, sem_ub, sem_vb)
        compute(lc0 + 1, u_b, v_b)
        return carry

    lax.fori_loop(0, n // 2 - 1, pair_body, 0)

    unpack_idx(start + n - 1, su_b, sv_b)
    issue(su_b, sv_b, u_b, v_b, sem_ub, sem_vb)
    wait(su_a, sv_a, u_a, v_a, sem_ua, sem_va)
    compute(n - 2, u_a, v_a)
    wait(su_b, sv_b, u_b, v_b, sem_ub, sem_vb)
    compute(n - 1, u_b, v_b)

    @pl.when(cid == 0)
    def _():
        pltpu.sync_copy(scores.at[pl.ds(0, CPW0 * CHUNK)],
                        out_hbm.at[sid, pl.ds(0, CPW0 * CHUNK)])

    @pl.when(cid == 1)
    def _():
        pltpu.sync_copy(scores.at[pl.ds(0, CPW1 * CHUNK)],
                        out_hbm.at[sid, pl.ds(CPW0 * CHUNK, CPW1 * CHUNK)])


def kernel(x, edge_index):
    xb = jnp.pad(x, ((0, N_PAD - N), (0, 0))).astype(jnp.bfloat16)
    xw = jax.lax.bitcast_convert_type(xb.reshape(N_PAD, DW, 2), jnp.int32)
    ei = edge_index.astype(jnp.int32)
    src = jnp.pad(ei[0], (0, E_PAD - E))
    dst = jnp.pad(ei[1], (0, E_PAD - E))
    packed = (src | (dst << 16)).reshape(NS, CPT, CHUNK)
    return _sc_dot(xw, packed).reshape(E_PAD)[:E]


# speedup vs baseline: 1.8568x; 1.0243x over previous
import functools

import jax
import jax.numpy as jnp
from jax import lax
from jax.experimental import pallas as pl
from jax.experimental.pallas import tpu as pltpu
from jax.experimental.pallas import tpu_sc as plsc

NC, NS, L = 2, 16, 16
NW = NC * NS
E = 320000
N = 10000
N_PAD = 10240
D = 128
DW = D // 2
CHUNK = 128
CPW = 80
EPW = CPW * CHUNK
E_PAD = NW * EPW
SEGS = DW // L
GROUPS = CHUNK // L
ROWS_PER_TILE = N_PAD // NS

_mesh = plsc.VectorSubcoreMesh(core_axis_name="c", subcore_axis_name="s")


@functools.partial(
    pl.kernel,
    out_type=jax.ShapeDtypeStruct((NW, EPW), jnp.float32),
    mesh=_mesh,
    compiler_params=pltpu.CompilerParams(needs_layout_passes=False,
                                         use_tc_tiling_on_sc=False),
    scratch_types=[
        pltpu.VMEM((CPW, CHUNK), jnp.int32),
        pltpu.VMEM((CHUNK,), jnp.int32),
        pltpu.VMEM((CHUNK,), jnp.int32),
        pltpu.VMEM((CHUNK,), jnp.int32),
        pltpu.VMEM((CHUNK,), jnp.int32),
        pltpu.VMEM((CHUNK, DW), jnp.int32),
        pltpu.VMEM((CHUNK, DW), jnp.int32),
        pltpu.VMEM((CHUNK, DW), jnp.int32),
        pltpu.VMEM((CHUNK, DW), jnp.int32),
        pltpu.VMEM((EPW,), jnp.float32),
        pltpu.SemaphoreType.DMA,
        pltpu.SemaphoreType.DMA,
        pltpu.SemaphoreType.DMA,
        pltpu.SemaphoreType.DMA,
    ],
)
def _sc_dot(x_hbm, pidx_hbm, out_hbm,
            pidx, su_a, su_b, sv_a, sv_b, u_a, u_b, v_a, v_b, scores,
            sem_ua, sem_ub, sem_va, sem_vb):
    cid = lax.axis_index("c")
    sid = lax.axis_index("s")
    wid = sid * NC + cid
    lanes = lax.iota(jnp.int32, L)

    pltpu.sync_copy(pidx_hbm.at[wid], pidx)

    def unpack_idx(c, subuf, svbuf):
        def body(g, carry):
            w = pidx[c, pl.ds(g * L, L)]
            subuf[pl.ds(g * L, L)] = w & 0xFFFF
            svbuf[pl.ds(g * L, L)] = lax.shift_right_logical(w, 16)
            return carry
        lax.fori_loop(0, GROUPS, body, 0)

    def issue(subuf, svbuf, ubuf, vbuf, sem_u, sem_v):
        pltpu.async_copy(x_hbm.at[subuf], ubuf, sem_u)
        pltpu.async_copy(x_hbm.at[svbuf], vbuf, sem_v)

    def wait(subuf, svbuf, ubuf, vbuf, sem_u, sem_v):
        pltpu.make_async_copy(x_hbm.at[subuf], ubuf, sem_u).wait()
        pltpu.make_async_copy(x_hbm.at[svbuf], vbuf, sem_v).wait()

    def compute(c, ubuf, vbuf):
        def group_body(g, gcarry):
            vec = jnp.zeros((L,), jnp.float32)
            for e in range(L):
                row = g * L + e
                acc = jnp.zeros((L,), jnp.float32)
                for s in range(SEGS):
                    useg = plsc.bitcast(ubuf[row, pl.ds(s * L, L)],
                                        jnp.bfloat16)
                    vseg = plsc.bitcast(vbuf[row, pl.ds(s * L, L)],
                                        jnp.bfloat16)
                    ua, ub = plsc.unpack(useg,
                                         format=plsc.PackFormat.INTERLEAVED)
                    va, vb = plsc.unpack(vseg,
                                         format=plsc.PackFormat.INTERLEAVED)
                    acc = acc + ua * va + ub * vb
                vec = jnp.where(lanes == e, jnp.sum(acc), vec)
            scores[pl.ds(c * CHUNK + g * L, L)] = vec
            return gcarry

        lax.fori_loop(0, GROUPS, group_body, 0)

    unpack_idx(0, su_a, sv_a)
    issue(su_a, sv_a, u_a, v_a, sem_ua, sem_va)

    def pair_body(i, carry):
        c0 = 2 * i
        unpack_idx(c0 + 1, su_b, sv_b)
        issue(su_b, sv_b, u_b, v_b, sem_ub, sem_vb)
        wait(su_a, sv_a, u_a, v_a, sem_ua, sem_va)
        compute(c0, u_a, v_a)
        unpack_idx(c0 + 2, su_a, sv_a)
        issue(su_a, sv_a, u_a, v_a, sem_ua, sem_va)
        wait(su_b, sv_b, u_b, v_b, sem_ub, sem_vb)
        compute(c0 + 1, u_b, v_b)
        return carry

    lax.fori_loop(0, CPW // 2 - 1, pair_body, 0)

    unpack_idx(CPW - 1, su_b, sv_b)
    issue(su_b, sv_b, u_b, v_b, sem_ub, sem_vb)
    wait(su_a, sv_a, u_a, v_a, sem_ua, sem_va)
    compute(CPW - 2, u_a, v_a)
    wait(su_b, sv_b, u_b, v_b, sem_ub, sem_vb)
    compute(CPW - 1, u_b, v_b)

    pltpu.sync_copy(scores, out_hbm.at[wid])


def kernel(x, edge_index):
    xb = jnp.pad(x, ((0, N_PAD - N), (0, 0))).astype(jnp.bfloat16)
    xw = jax.lax.bitcast_convert_type(xb.reshape(N_PAD, DW, 2), jnp.int32)
    ei = edge_index.astype(jnp.int32)
    src = jnp.pad(ei[0], (0, E_PAD - E))
    dst = jnp.pad(ei[1], (0, E_PAD - E))
    packed = (src | (dst << 16)).reshape(NW, CPW, CHUNK)
    return _sc_dot(xw, packed).reshape(E_PAD)[:E]
